# baseline (speedup 1.0000x reference)
import functools

import jax
import jax.numpy as jnp
from jax import lax
from jax.experimental import pallas as pl
from jax.experimental.pallas import tpu as pltpu
from jax.experimental.pallas import tpu_sc as plsc

N_BINS = 15
N_ROWS = 500000
N_CLASSES = 100

BLOCK_ROWS = 4096
GRID = -(-N_ROWS // BLOCK_ROWS)
NPAD = GRID * BLOCK_ROWS

N_TILES = 16
CHUNK = NPAD // N_TILES
NVEC = CHUNK // 16


def _rowstats_body(logits_t_ref, labels_ref, conf_ref, acc_ref):
    x = logits_t_ref[...]
    m = jnp.max(x, axis=0, keepdims=True)
    s = jnp.sum(jnp.exp(x - m), axis=0)
    conf_ref[...] = 1.0 / s
    row = lax.broadcasted_iota(jnp.int32, x.shape, 0)
    pred = jnp.min(jnp.where(x == m, row, N_CLASSES), axis=0)
    acc_ref[...] = (pred == labels_ref[...]).astype(jnp.float32)


_rowstats = pl.pallas_call(
    _rowstats_body,
    grid=(GRID,),
    in_specs=[
        pl.BlockSpec((N_CLASSES, BLOCK_ROWS), lambda i: (0, i)),
        pl.BlockSpec((BLOCK_ROWS,), lambda i: (i,)),
    ],
    out_specs=[
        pl.BlockSpec((BLOCK_ROWS,), lambda i: (i,)),
        pl.BlockSpec((BLOCK_ROWS,), lambda i: (i,)),
    ],
    out_shape=[
        jax.ShapeDtypeStruct((NPAD,), jnp.float32),
        jax.ShapeDtypeStruct((NPAD,), jnp.float32),
    ],
    compiler_params=pltpu.CompilerParams(
        dimension_semantics=("arbitrary",),
    ),
)


def _sc_hist_body(conf_hbm, acc_hbm, out_hbm, conf_v, acc_v, acc3_a, part_v,
                  shared, allpart_v, out_v):
    wid = lax.axis_index("s")
    base = wid * CHUNK
    pltpu.sync_copy(conf_hbm.at[pl.ds(base, CHUNK)], conf_v)
    pltpu.sync_copy(acc_hbm.at[pl.ds(base, CHUNK)], acc_v)

    zeros16 = jnp.zeros((16,), jnp.float32)
    for r in range(48):
        acc3_a[pl.ds(r * 16, 16)] = zeros16

    lane16 = lax.iota(jnp.int32, 16) * 16
    lane = lax.iota(jnp.int32, 16)
    ones16 = jnp.ones((16,), jnp.float32)
    nvec = jnp.clip((N_ROWS - base) // 16, 0, NVEC)

    @plsc.parallel_loop(0, nvec, 1, unroll=8)
    def _(i):
        off = i * 16
        cv = conf_v[pl.ds(off, 16)]
        av = acc_v[pl.ds(off, 16)]
        b = jnp.minimum((cv * float(N_BINS)).astype(jnp.int32), N_BINS - 1)
        idx = lane16 + b
        plsc.addupdate_scatter(acc3_a, [idx], ones16)
        plsc.addupdate_scatter(acc3_a, [idx + 256], cv)
        plsc.addupdate_scatter(acc3_a, [idx + 512], av)

    for j in range(3):
        v = acc3_a[pl.ds(j * 256, 16)]
        for r in range(1, 16):
            v = v + acc3_a[pl.ds(j * 256 + r * 16, 16)]
        part_v[pl.ds(j * 16, 16)] = v
    part_v[pl.ds(48, 16)] = zeros16
    pltpu.sync_copy(part_v, shared.at[pl.ds(wid * 64, 64)])
    plsc.subcore_barrier()

    @pl.when(wid == 0)
    def _():
        pltpu.sync_copy(shared, allpart_v)
        res = []
        for j in range(3):
            v = allpart_v[pl.ds(j * 16, 16)]
            for w in range(1, N_TILES):
                v = v + allpart_v[pl.ds(w * 64 + j * 16, 16)]
            res.append(v)
        cnt, sconf, sacc = res
        safe = jnp.maximum(cnt, 1.0)
        gap = jnp.abs(sconf / safe - sacc / safe) * (cnt * (1.0 / N_ROWS))
        gap = jnp.where(cnt > 0.0, gap, 0.0)
        ece = jnp.sum(gap)
        out_v[...] = jnp.where(lane == 0, ece, 0.0)
        pltpu.sync_copy(out_v, out_hbm)


@functools.cache
def _make_sc_hist():
    mesh = plsc.VectorSubcoreMesh(
        core_axis_name="c", subcore_axis_name="s", num_cores=1, num_subcores=16
    )
    return pl.kernel(
        _sc_hist_body,
        out_type=jax.ShapeDtypeStruct((16,), jnp.float32),
        mesh=mesh,
        compiler_params=pltpu.CompilerParams(needs_layout_passes=False),
        scratch_types=[
            pltpu.VMEM((CHUNK,), jnp.float32),
            pltpu.VMEM((CHUNK,), jnp.float32),
            pltpu.VMEM((768,), jnp.float32),
            pltpu.VMEM((64,), jnp.float32),
            pltpu.VMEM_SHARED((N_TILES * 64,), jnp.float32),
            pltpu.VMEM((N_TILES * 64,), jnp.float32),
            pltpu.VMEM((16,), jnp.float32),
        ],
    )


def kernel(logits, labels):
    conf, acc = _rowstats(logits.T, labels)
    ece16 = _make_sc_hist()(conf, acc)
    return ece16[0:1]


# speedup vs baseline: 1.0018x; 1.0018x over previous
import functools

import jax
import jax.numpy as jnp
from jax import lax
from jax.experimental import pallas as pl
from jax.experimental.pallas import tpu as pltpu
from jax.experimental.pallas import tpu_sc as plsc

N_BINS = 15
N_ROWS = 500000
N_CLASSES = 100

BLOCK_ROWS = 4096
GRID = -(-N_ROWS // BLOCK_ROWS)
NPAD = GRID * BLOCK_ROWS

N_TILES = 16
CHUNK = NPAD // N_TILES
NVEC = CHUNK // 16


def _rowstats_body(logits_t_ref, labels_ref, conf_ref, acc_ref):
    x = logits_t_ref[...]
    m = jnp.max(x, axis=0, keepdims=True)
    s = jnp.sum(jnp.exp(x - m), axis=0)
    conf_ref[...] = 1.0 / s
    row = lax.broadcasted_iota(jnp.int32, x.shape, 0)
    pred = jnp.min(jnp.where(x == m, row, N_CLASSES), axis=0)
    acc_ref[...] = (pred == labels_ref[...]).astype(jnp.float32)


_rowstats = pl.pallas_call(
    _rowstats_body,
    grid=(GRID,),
    in_specs=[
        pl.BlockSpec((N_CLASSES, BLOCK_ROWS), lambda i: (0, i)),
        pl.BlockSpec((BLOCK_ROWS,), lambda i: (i,)),
    ],
    out_specs=[
        pl.BlockSpec((BLOCK_ROWS,), lambda i: (i,)),
        pl.BlockSpec((BLOCK_ROWS,), lambda i: (i,)),
    ],
    out_shape=[
        jax.ShapeDtypeStruct((NPAD,), jnp.float32),
        jax.ShapeDtypeStruct((NPAD,), jnp.float32),
    ],
    compiler_params=pltpu.CompilerParams(
        dimension_semantics=("arbitrary",),
    ),
)


def _sc_hist_body(conf_hbm, acc_hbm, out_hbm, conf_v, acc_v, acc3_a, part_v,
                  shared, allpart_v, out_v):
    wid = lax.axis_index("s")
    base = wid * CHUNK
    pltpu.sync_copy(conf_hbm.at[pl.ds(base, CHUNK)], conf_v)
    pltpu.sync_copy(acc_hbm.at[pl.ds(base, CHUNK)], acc_v)

    zeros16 = jnp.zeros((16,), jnp.float32)
    for r in range(48):
        acc3_a[pl.ds(r * 16, 16)] = zeros16

    lane16 = lax.iota(jnp.int32, 16) * 16
    lane = lax.iota(jnp.int32, 16)
    ones16 = jnp.ones((16,), jnp.float32)
    nvec = jnp.clip((N_ROWS - base) // 16, 0, NVEC)

    @plsc.parallel_loop(0, nvec, 1, unroll=4)
    def _(i):
        off = i * 16
        cv = conf_v[pl.ds(off, 16)]
        av = acc_v[pl.ds(off, 16)]
        b = jnp.minimum((cv * float(N_BINS)).astype(jnp.int32), N_BINS - 1)
        idx = lane16 + b
        plsc.addupdate_scatter(acc3_a, [idx], ones16)
        plsc.addupdate_scatter(acc3_a, [idx + 256], cv)
        plsc.addupdate_scatter(acc3_a, [idx + 512], av)

    for j in range(3):
        v = acc3_a[pl.ds(j * 256, 16)]
        for r in range(1, 16):
            v = v + acc3_a[pl.ds(j * 256 + r * 16, 16)]
        part_v[pl.ds(j * 16, 16)] = v
    part_v[pl.ds(48, 16)] = zeros16
    pltpu.sync_copy(part_v, shared.at[pl.ds(wid * 64, 64)])
    plsc.subcore_barrier()

    @pl.when(wid == 0)
    def _():
        pltpu.sync_copy(shared, allpart_v)
        res = []
        for j in range(3):
            v = allpart_v[pl.ds(j * 16, 16)]
            for w in range(1, N_TILES):
                v = v + allpart_v[pl.ds(w * 64 + j * 16, 16)]
            res.append(v)
        cnt, sconf, sacc = res
        safe = jnp.maximum(cnt, 1.0)
        gap = jnp.abs(sconf / safe - sacc / safe) * (cnt * (1.0 / N_ROWS))
        gap = jnp.where(cnt > 0.0, gap, 0.0)
        ece = jnp.sum(gap)
        out_v[...] = jnp.where(lane == 0, ece, 0.0)
        pltpu.sync_copy(out_v, out_hbm)


@functools.cache
def _make_sc_hist():
    mesh = plsc.VectorSubcoreMesh(
        core_axis_name="c", subcore_axis_name="s", num_cores=1, num_subcores=16
    )
    return pl.kernel(
        _sc_hist_body,
        out_type=jax.ShapeDtypeStruct((16,), jnp.float32),
        mesh=mesh,
        compiler_params=pltpu.CompilerParams(needs_layout_passes=False),
        scratch_types=[
            pltpu.VMEM((CHUNK,), jnp.float32),
            pltpu.VMEM((CHUNK,), jnp.float32),
            pltpu.VMEM((768,), jnp.float32),
            pltpu.VMEM((64,), jnp.float32),
            pltpu.VMEM_SHARED((N_TILES * 64,), jnp.float32),
            pltpu.VMEM((N_TILES * 64,), jnp.float32),
            pltpu.VMEM((16,), jnp.float32),
        ],
    )


def kernel(logits, labels):
    conf, acc = _rowstats(logits.T, labels)
    ece16 = _make_sc_hist()(conf, acc)
    return ece16[0:1]


# speedup vs baseline: 1.0535x; 1.0516x over previous
import functools

import jax
import jax.numpy as jnp
from jax import lax
from jax.experimental import pallas as pl
from jax.experimental.pallas import tpu as pltpu
from jax.experimental.pallas import tpu_sc as plsc

N_BINS = 15
N_ROWS = 500000
N_CLASSES = 100

BLOCK_ROWS = 4096
GRID = -(-N_ROWS // BLOCK_ROWS)
GRID_A = 62
GRID_B = GRID - GRID_A
NPAD_A = GRID_A * BLOCK_ROWS
NPAD_B = GRID_B * BLOCK_ROWS

N_TILES = 16


def _rowstats_body(logits_t_ref, labels_ref, conf_ref, acc_ref):
    x = logits_t_ref[...]
    m = jnp.max(x, axis=0, keepdims=True)
    s = jnp.sum(jnp.exp(x - m), axis=0)
    conf_ref[...] = 1.0 / s
    row = lax.broadcasted_iota(jnp.int32, x.shape, 0)
    pred = jnp.min(jnp.where(x == m, row, N_CLASSES), axis=0)
    acc_ref[...] = (pred == labels_ref[...]).astype(jnp.float32)


def _make_rowstats(grid, block_off, npad):
    return pl.pallas_call(
        _rowstats_body,
        grid=(grid,),
        in_specs=[
            pl.BlockSpec((N_CLASSES, BLOCK_ROWS), lambda i: (0, i + block_off)),
            pl.BlockSpec((BLOCK_ROWS,), lambda i: (i + block_off,)),
        ],
        out_specs=[
            pl.BlockSpec((BLOCK_ROWS,), lambda i: (i,)),
            pl.BlockSpec((BLOCK_ROWS,), lambda i: (i,)),
        ],
        out_shape=[
            jax.ShapeDtypeStruct((npad,), jnp.float32),
            jax.ShapeDtypeStruct((npad,), jnp.float32),
        ],
        compiler_params=pltpu.CompilerParams(
            dimension_semantics=("arbitrary",),
        ),
    )


_rowstats_a = _make_rowstats(GRID_A, 0, NPAD_A)
_rowstats_b = _make_rowstats(GRID_B, GRID_A, NPAD_B)


def _sc_hist_body(npad, row_base, final, *refs):
    if final:
        (conf_hbm, acc_hbm, prev_hbm, out_hbm, conf_v, acc_v, acc3_a,
         part_v, shared, allpart_v, prev_v, out_v) = refs
    else:
        (conf_hbm, acc_hbm, out_hbm, conf_v, acc_v, acc3_a,
         part_v, shared, allpart_v, out_v) = refs
    chunk = npad // N_TILES
    max_nvec = chunk // 16
    wid = lax.axis_index("s")
    base = wid * chunk
    pltpu.sync_copy(conf_hbm.at[pl.ds(base, chunk)], conf_v)
    pltpu.sync_copy(acc_hbm.at[pl.ds(base, chunk)], acc_v)

    zeros16 = jnp.zeros((16,), jnp.float32)
    for r in range(48):
        acc3_a[pl.ds(r * 16, 16)] = zeros16

    lane16 = lax.iota(jnp.int32, 16) * 16
    lane = lax.iota(jnp.int32, 16)
    ones16 = jnp.ones((16,), jnp.float32)
    nvec = jnp.clip((N_ROWS - row_base - base) // 16, 0, max_nvec)

    @plsc.parallel_loop(0, nvec, 1, unroll=4)
    def _(i):
        off = i * 16
        cv = conf_v[pl.ds(off, 16)]
        av = acc_v[pl.ds(off, 16)]
        b = jnp.minimum((cv * float(N_BINS)).astype(jnp.int32), N_BINS - 1)
        idx = lane16 + b
        plsc.addupdate_scatter(acc3_a, [idx], ones16)
        plsc.addupdate_scatter(acc3_a, [idx + 256], cv)
        plsc.addupdate_scatter(acc3_a, [idx + 512], av)

    for j in range(3):
        v = acc3_a[pl.ds(j * 256, 16)]
        for r in range(1, 16):
            v = v + acc3_a[pl.ds(j * 256 + r * 16, 16)]
        part_v[pl.ds(j * 16, 16)] = v
    part_v[pl.ds(48, 16)] = zeros16
    pltpu.sync_copy(part_v, shared.at[pl.ds(wid * 64, 64)])
    plsc.subcore_barrier()

    @pl.when(wid == 0)
    def _():
        pltpu.sync_copy(shared, allpart_v)
        if final:
            pltpu.sync_copy(prev_hbm, prev_v)
        res = []
        for j in range(3):
            v = allpart_v[pl.ds(j * 16, 16)]
            for w in range(1, N_TILES):
                v = v + allpart_v[pl.ds(w * 64 + j * 16, 16)]
            if final:
                v = v + prev_v[pl.ds(j * 16, 16)]
            res.append(v)
        cnt, sconf, sacc = res
        if final:
            safe = jnp.maximum(cnt, 1.0)
            gap = jnp.abs(sconf / safe - sacc / safe) * (cnt * (1.0 / N_ROWS))
            gap = jnp.where(cnt > 0.0, gap, 0.0)
            ece = jnp.sum(gap)
            out_v[...] = jnp.where(lane == 0, ece, 0.0)
            pltpu.sync_copy(out_v, out_hbm)
        else:
            out_v[pl.ds(0, 16)] = cnt
            out_v[pl.ds(16, 16)] = sconf
            out_v[pl.ds(32, 16)] = sacc
            out_v[pl.ds(48, 16)] = zeros16
            pltpu.sync_copy(out_v, out_hbm)


@functools.cache
def _make_sc_hist(npad, row_base, final):
    mesh = plsc.VectorSubcoreMesh(
        core_axis_name="c", subcore_axis_name="s", num_cores=1, num_subcores=16
    )
    chunk = npad // N_TILES
    scratch = [
        pltpu.VMEM((chunk,), jnp.float32),
        pltpu.VMEM((chunk,), jnp.float32),
        pltpu.VMEM((768,), jnp.float32),
        pltpu.VMEM((64,), jnp.float32),
        pltpu.VMEM_SHARED((N_TILES * 64,), jnp.float32),
        pltpu.VMEM((N_TILES * 64,), jnp.float32),
    ]
    if final:
        scratch.append(pltpu.VMEM((64,), jnp.float32))
        scratch.append(pltpu.VMEM((16,), jnp.float32))
        out_type = jax.ShapeDtypeStruct((16,), jnp.float32)
    else:
        scratch.append(pltpu.VMEM((64,), jnp.float32))
        out_type = jax.ShapeDtypeStruct((64,), jnp.float32)
    return pl.kernel(
        functools.partial(_sc_hist_body, npad, row_base, final),
        out_type=out_type,
        mesh=mesh,
        compiler_params=pltpu.CompilerParams(needs_layout_passes=False),
        scratch_types=scratch,
    )


def kernel(logits, labels):
    lt = logits.T
    conf_a, acc_a = _rowstats_a(lt, labels)
    conf_b, acc_b = _rowstats_b(lt, labels)
    part_a = _make_sc_hist(NPAD_A, 0, False)(conf_a, acc_a)
    ece16 = _make_sc_hist(NPAD_B, NPAD_A, True)(conf_b, acc_b, part_a)
    return ece16[0:1]


# speedup vs baseline: 1.0973x; 1.0416x over previous
import functools

import jax
import jax.numpy as jnp
from jax import lax
from jax.experimental import pallas as pl
from jax.experimental.pallas import tpu as pltpu
from jax.experimental.pallas import tpu_sc as plsc

N_BINS = 15
N_ROWS = 500000
N_CLASSES = 100

BLOCK_ROWS = 4096
GRID = -(-N_ROWS // BLOCK_ROWS)
GRID_A = 102
GRID_B = GRID - GRID_A
NPAD_A = GRID_A * BLOCK_ROWS
NPAD_B = GRID_B * BLOCK_ROWS

N_TILES = 16


def _rowstats_body(logits_t_ref, labels_ref, conf_ref, acc_ref):
    x = logits_t_ref[...]
    m = jnp.max(x, axis=0, keepdims=True)
    s = jnp.sum(jnp.exp(x - m), axis=0)
    conf_ref[...] = 1.0 / s
    row = lax.broadcasted_iota(jnp.int32, x.shape, 0)
    pred = jnp.min(jnp.where(x == m, row, N_CLASSES), axis=0)
    acc_ref[...] = (pred == labels_ref[...]).astype(jnp.float32)


def _make_rowstats(grid, block_off, npad):
    return pl.pallas_call(
        _rowstats_body,
        grid=(grid,),
        in_specs=[
            pl.BlockSpec((N_CLASSES, BLOCK_ROWS), lambda i: (0, i + block_off)),
            pl.BlockSpec((BLOCK_ROWS,), lambda i: (i + block_off,)),
        ],
        out_specs=[
            pl.BlockSpec((BLOCK_ROWS,), lambda i: (i,)),
            pl.BlockSpec((BLOCK_ROWS,), lambda i: (i,)),
        ],
        out_shape=[
            jax.ShapeDtypeStruct((npad,), jnp.float32),
            jax.ShapeDtypeStruct((npad,), jnp.float32),
        ],
        compiler_params=pltpu.CompilerParams(
            dimension_semantics=("arbitrary",),
        ),
    )


_rowstats_a = _make_rowstats(GRID_A, 0, NPAD_A)
_rowstats_b = _make_rowstats(GRID_B, GRID_A, NPAD_B)


def _sc_hist_body(npad, row_base, final, *refs):
    if final:
        (conf_hbm, acc_hbm, prev_hbm, out_hbm, conf_v, acc_v, acc3_a,
         part_v, shared, allpart_v, prev_v, out_v) = refs
    else:
        (conf_hbm, acc_hbm, out_hbm, conf_v, acc_v, acc3_a,
         part_v, shared, allpart_v, out_v) = refs
    chunk = npad // N_TILES
    max_nvec = chunk // 16
    wid = lax.axis_index("s")
    base = wid * chunk
    pltpu.sync_copy(conf_hbm.at[pl.ds(base, chunk)], conf_v)
    pltpu.sync_copy(acc_hbm.at[pl.ds(base, chunk)], acc_v)

    zeros16 = jnp.zeros((16,), jnp.float32)
    for r in range(48):
        acc3_a[pl.ds(r * 16, 16)] = zeros16

    lane16 = lax.iota(jnp.int32, 16) * 16
    lane = lax.iota(jnp.int32, 16)
    ones16 = jnp.ones((16,), jnp.float32)
    nvec = jnp.clip((N_ROWS - row_base - base) // 16, 0, max_nvec)

    @plsc.parallel_loop(0, nvec, 1, unroll=4)
    def _(i):
        off = i * 16
        cv = conf_v[pl.ds(off, 16)]
        av = acc_v[pl.ds(off, 16)]
        b = jnp.minimum((cv * float(N_BINS)).astype(jnp.int32), N_BINS - 1)
        idx = lane16 + b
        plsc.addupdate_scatter(acc3_a, [idx], ones16)
        plsc.addupdate_scatter(acc3_a, [idx + 256], cv)
        plsc.addupdate_scatter(acc3_a, [idx + 512], av)

    for j in range(3):
        v = acc3_a[pl.ds(j * 256, 16)]
        for r in range(1, 16):
            v = v + acc3_a[pl.ds(j * 256 + r * 16, 16)]
        part_v[pl.ds(j * 16, 16)] = v
    part_v[pl.ds(48, 16)] = zeros16
    pltpu.sync_copy(part_v, shared.at[pl.ds(wid * 64, 64)])
    plsc.subcore_barrier()

    @pl.when(wid == 0)
    def _():
        pltpu.sync_copy(shared, allpart_v)
        if final:
            pltpu.sync_copy(prev_hbm, prev_v)
        res = []
        for j in range(3):
            v = allpart_v[pl.ds(j * 16, 16)]
            for w in range(1, N_TILES):
                v = v + allpart_v[pl.ds(w * 64 + j * 16, 16)]
            if final:
                v = v + prev_v[pl.ds(j * 16, 16)]
            res.append(v)
        cnt, sconf, sacc = res
        if final:
            safe = jnp.maximum(cnt, 1.0)
            gap = jnp.abs(sconf / safe - sacc / safe) * (cnt * (1.0 / N_ROWS))
            gap = jnp.where(cnt > 0.0, gap, 0.0)
            ece = jnp.sum(gap)
            out_v[...] = jnp.where(lane == 0, ece, 0.0)
            pltpu.sync_copy(out_v, out_hbm)
        else:
            out_v[pl.ds(0, 16)] = cnt
            out_v[pl.ds(16, 16)] = sconf
            out_v[pl.ds(32, 16)] = sacc
            out_v[pl.ds(48, 16)] = zeros16
            pltpu.sync_copy(out_v, out_hbm)


@functools.cache
def _make_sc_hist(npad, row_base, final):
    mesh = plsc.VectorSubcoreMesh(
        core_axis_name="c", subcore_axis_name="s", num_cores=1, num_subcores=16
    )
    chunk = npad // N_TILES
    scratch = [
        pltpu.VMEM((chunk,), jnp.float32),
        pltpu.VMEM((chunk,), jnp.float32),
        pltpu.VMEM((768,), jnp.float32),
        pltpu.VMEM((64,), jnp.float32),
        pltpu.VMEM_SHARED((N_TILES * 64,), jnp.float32),
        pltpu.VMEM((N_TILES * 64,), jnp.float32),
    ]
    if final:
        scratch.append(pltpu.VMEM((64,), jnp.float32))
        scratch.append(pltpu.VMEM((16,), jnp.float32))
        out_type = jax.ShapeDtypeStruct((16,), jnp.float32)
    else:
        scratch.append(pltpu.VMEM((64,), jnp.float32))
        out_type = jax.ShapeDtypeStruct((64,), jnp.float32)
    return pl.kernel(
        functools.partial(_sc_hist_body, npad, row_base, final),
        out_type=out_type,
        mesh=mesh,
        compiler_params=pltpu.CompilerParams(needs_layout_passes=False),
        scratch_types=scratch,
    )


def kernel(logits, labels):
    lt = logits.T
    conf_a, acc_a = _rowstats_a(lt, labels)
    conf_b, acc_b = _rowstats_b(lt, labels)
    part_a = _make_sc_hist(NPAD_A, 0, False)(conf_a, acc_a)
    ece16 = _make_sc_hist(NPAD_B, NPAD_A, True)(conf_b, acc_b, part_a)
    return ece16[0:1]
